# baseline (speedup 1.0000x reference)
import functools

import jax
import jax.numpy as jnp
from jax import lax
from jax.experimental import pallas as pl
from jax.experimental.pallas import tpu as pltpu
from jax.experimental.pallas import tpu_sc as plsc

VOCAB = 100000
D = 64
B = 1024
MAX_NORM = 1.0

NC, NS = 2, 16
NW = NC * NS
BPW = B // NW

BN = 2048
NFULL = VOCAB // BN
REM = VOCAB - NFULL * BN
NSTEP = NFULL + 1
K = 3


_sc_mesh = plsc.VectorSubcoreMesh(
    core_axis_name="c", subcore_axis_name="s", num_cores=NC, num_subcores=NS
)


@functools.partial(
    pl.kernel,
    out_type=jax.ShapeDtypeStruct((B, D), jnp.float32),
    mesh=_sc_mesh,
    scratch_types=[
        pltpu.VMEM((BPW,), jnp.int32),
        pltpu.VMEM((BPW, D), jnp.float32),
        pltpu.SemaphoreType.DMA,
    ],
    compiler_params=pltpu.CompilerParams(use_tc_tiling_on_sc=False),
)
def _sc_gather(table_hbm, idx_hbm, out_hbm, idx_v, rows_v, sem):
    wid = lax.axis_index("s") * NC + lax.axis_index("c")
    base = wid * BPW
    pltpu.sync_copy(idx_hbm.at[pl.ds(base, BPW)], idx_v)
    pltpu.async_copy(table_hbm.at[idx_v], rows_v, sem).wait()
    pltpu.sync_copy(rows_v, out_hbm.at[pl.ds(base, BPW)])


def _proj_body(x_ref, w_ref, b_ref, o_hbm, obuf, tbuf, sems):
    j = pl.program_id(0)
    x = x_ref[...]
    sq = jnp.sum(x * x, axis=1, keepdims=True)
    nrm = jnp.sqrt(sq)
    scale = jnp.where(nrm > MAX_NORM, MAX_NORM / (nrm + 1e-7), 1.0)
    xn = x * scale
    acc = lax.dot_general(
        xn, w_ref[...], (((1,), (1,)), ((), ())),
        preferred_element_type=jnp.float32,
    )
    res = acc + b_ref[...]

    for k in range(K):
        @pl.when(lax.rem(j, K) == k)
        def _(k=k):
            @pl.when(j >= K)
            def _():
                pltpu.make_async_copy(
                    obuf.at[k], o_hbm.at[:, pl.ds((j - K) * BN, BN)], sems.at[k]
                ).wait()
            @pl.when(j < NFULL)
            def _():
                obuf[k] = res
                pltpu.async_copy(
                    obuf.at[k], o_hbm.at[:, pl.ds(j * BN, BN)], sems.at[k],
                    priority=k % 2,
                )

    @pl.when(j == NFULL)
    def _():
        tbuf[...] = res[:, :REM]
        pltpu.async_copy(
            tbuf, o_hbm.at[:, pl.ds(NFULL * BN, REM)], sems.at[K], priority=0
        )
        for jj in range(NSTEP - K, NFULL):
            pltpu.make_async_copy(
                obuf.at[jj % K], o_hbm.at[:, pl.ds(jj * BN, BN)],
                sems.at[jj % K],
            ).wait()
        pltpu.make_async_copy(
            tbuf, o_hbm.at[:, pl.ds(NFULL * BN, REM)], sems.at[K]
        ).wait()


_proj = pl.pallas_call(
    _proj_body,
    grid=(NSTEP,),
    in_specs=[
        pl.BlockSpec((B, D), lambda j: (0, 0)),
        pl.BlockSpec((BN, D), lambda j: (j, 0)),
        pl.BlockSpec((1, BN), lambda j: (0, j)),
    ],
    out_specs=pl.BlockSpec(memory_space=pl.ANY),
    out_shape=jax.ShapeDtypeStruct((B, VOCAB), jnp.float32),
    scratch_shapes=[
        pltpu.VMEM((K, B, BN), jnp.float32),
        pltpu.VMEM((B, REM), jnp.float32),
        pltpu.SemaphoreType.DMA((K + 1,)),
    ],
    compiler_params=pltpu.CompilerParams(
        dimension_semantics=("arbitrary",),
        vmem_limit_bytes=100 * 1024 * 1024,
    ),
)


@jax.jit
def kernel(inputs_, emb_table, lin_w, lin_b):
    idx = inputs_.astype(jnp.int32)
    x = _sc_gather(emb_table, idx)
    return _proj(x, lin_w, lin_b.reshape(1, VOCAB))


# speedup vs baseline: 1.0260x; 1.0260x over previous
import functools

import jax
import jax.numpy as jnp
from jax import lax
from jax.experimental import pallas as pl
from jax.experimental.pallas import tpu as pltpu
from jax.experimental.pallas import tpu_sc as plsc

VOCAB = 100000
D = 64
B = 1024
MAX_NORM = 1.0

NC, NS = 2, 16
NW = NC * NS
BPW = B // NW

BN = 2048


_sc_mesh = plsc.VectorSubcoreMesh(
    core_axis_name="c", subcore_axis_name="s", num_cores=NC, num_subcores=NS
)


@functools.partial(
    pl.kernel,
    out_type=jax.ShapeDtypeStruct((B, D), jnp.float32),
    mesh=_sc_mesh,
    scratch_types=[
        pltpu.VMEM((BPW,), jnp.int32),
        pltpu.VMEM((BPW, D), jnp.float32),
        pltpu.SemaphoreType.DMA,
    ],
    compiler_params=pltpu.CompilerParams(use_tc_tiling_on_sc=False),
)
def _sc_gather(table_hbm, idx_hbm, out_hbm, idx_v, rows_v, sem):
    wid = lax.axis_index("s") * NC + lax.axis_index("c")
    base = wid * BPW
    pltpu.sync_copy(idx_hbm.at[pl.ds(base, BPW)], idx_v)
    pltpu.async_copy(table_hbm.at[idx_v], rows_v, sem).wait()
    pltpu.sync_copy(rows_v, out_hbm.at[pl.ds(base, BPW)])


def _proj_body(x_ref, w_ref, b_ref, o_ref):
    x = x_ref[...]
    sq = jnp.sum(x * x, axis=1, keepdims=True)
    nrm = jnp.sqrt(sq)
    scale = jnp.where(nrm > MAX_NORM, MAX_NORM / (nrm + 1e-7), 1.0)
    xn = x * scale
    acc = lax.dot_general(
        xn, w_ref[...], (((1,), (1,)), ((), ())),
        preferred_element_type=jnp.float32,
    )
    o_ref[...] = acc + b_ref[...]


_proj = pl.pallas_call(
    _proj_body,
    grid=(pl.cdiv(VOCAB, BN),),
    in_specs=[
        pl.BlockSpec(memory_space=pltpu.VMEM),
        pl.BlockSpec((BN, D), lambda j: (j, 0)),
        pl.BlockSpec((1, BN), lambda j: (0, j)),
    ],
    out_specs=pl.BlockSpec((B, BN), lambda j: (0, j)),
    out_shape=jax.ShapeDtypeStruct((B, VOCAB), jnp.float32),
    compiler_params=pltpu.CompilerParams(
        dimension_semantics=("arbitrary",),
        vmem_limit_bytes=100 * 1024 * 1024,
    ),
)


@jax.jit
def kernel(inputs_, emb_table, lin_w, lin_b):
    idx = inputs_.astype(jnp.int32)
    x = _sc_gather(emb_table, idx)
    return _proj(x, lin_w, lin_b.reshape(1, VOCAB))
